# baseline (speedup 1.0000x reference)
import functools

import jax
import jax.numpy as jnp
from jax import lax
from jax.experimental import pallas as pl
from jax.experimental.pallas import tpu as pltpu
from jax.experimental.pallas import tpu_sc as plsc

_B, _H, _W = 8, 384, 384
_NPIX = _H * _W
_NWORDS = _NPIX // 2
_TILES_PER_IMG = 4
_ROWS_PER_TILE = _H // _TILES_PER_IMG
_CHUNK_ROWS = 8
_CHUNK_PIX = _CHUNK_ROWS * _W
_NCHUNKS = _ROWS_PER_TILE // _CHUNK_ROWS
_GROUPS_PER_ROW = _W // 16


def _sc_body(packed_ref, dvf_ref, out_ref, img_v, dvf_v, out_v):
    cid = lax.axis_index("c")
    sid = lax.axis_index("s")
    wid = sid * 2 + cid
    b = wid // _TILES_PER_IMG
    q = wid % _TILES_PER_IMG
    row0 = q * _ROWS_PER_TILE

    pltpu.sync_copy(packed_ref.at[b], img_v)

    lane = lax.iota(jnp.int32, 16)
    lane2 = lane * 2
    lanef = lane.astype(jnp.float32)

    for ch in range(_NCHUNKS):
        crow = row0 + ch * _CHUNK_ROWS
        dsrc = (b * _NPIX + crow * _W) * 2
        pltpu.sync_copy(dvf_ref.at[pl.ds(dsrc, _CHUNK_PIX * 2)], dvf_v)

        def _row(r, carry):
            rowf = (crow + r).astype(jnp.float32)

            def _grp(t, carry2):
                p0 = r * _W + t * 16
                base2 = p0 * 2
                dx = plsc.load_gather(dvf_v, [base2 + lane2])
                dy = plsc.load_gather(dvf_v, [base2 + lane2 + 1])

                fx = (t * 16).astype(jnp.float32) + lanef + dx
                fy = rowf + dy
                x0 = fx.astype(jnp.int32)
                y0 = fy.astype(jnp.int32)
                x1 = x0 + 1
                y1 = y0 + 1
                x0 = jnp.clip(x0, 0, _W - 1)
                x1 = jnp.clip(x1, 0, _W - 1)
                y0 = jnp.clip(y0, 0, _H - 1)
                y1 = jnp.clip(y1, 0, _H - 1)

                ry0 = y0 * _W
                ry1 = y1 * _W
                pa = ry0 + x0
                pb = ry1 + x0
                pc = ry0 + x1
                pd = ry1 + x1

                def fetch(p):
                    word = plsc.load_gather(img_v, [p >> 1])
                    sh = ((p & 1) ^ 1) << 4
                    bits = (word << sh) & jnp.int32(-65536)
                    return plsc.bitcast(bits, jnp.float32)

                va = fetch(pa)
                vb = fetch(pb)
                vc = fetch(pc)
                vd = fetch(pd)

                x0f = x0.astype(jnp.float32)
                x1f = x1.astype(jnp.float32)
                y0f = y0.astype(jnp.float32)
                y1f = y1.astype(jnp.float32)
                wx1 = x1f - fx
                wx0 = fx - x0f
                wy1 = y1f - fy
                wy0 = fy - y0f
                res = ((wx1 * wy1) * va + (wx1 * wy0) * vb
                       + (wx0 * wy1) * vc + (wx0 * wy0) * vd)
                out_v[pl.ds(p0, 16)] = res
                return carry2

            return lax.fori_loop(0, _GROUPS_PER_ROW, _grp, carry)

        lax.fori_loop(0, _CHUNK_ROWS, _row, jnp.int32(0))

        dst = b * _NPIX + crow * _W
        pltpu.sync_copy(out_v, out_ref.at[pl.ds(dst, _CHUNK_PIX)])


@jax.jit
def _run(packed, dvf_flat):
    mesh = plsc.VectorSubcoreMesh(core_axis_name="c", subcore_axis_name="s")
    fn = pl.kernel(
        _sc_body,
        out_type=jax.ShapeDtypeStruct((_B * _NPIX,), jnp.float32),
        mesh=mesh,
        scratch_types=[
            pltpu.VMEM((_NWORDS,), jnp.int32),
            pltpu.VMEM((_CHUNK_PIX * 2,), jnp.float32),
            pltpu.VMEM((_CHUNK_PIX,), jnp.float32),
        ],
        compiler_params=pltpu.CompilerParams(needs_layout_passes=False),
    )
    return fn(packed, dvf_flat)


def kernel(imgs, dvfs):
    B, H, W, C = imgs.shape
    imgs_bf = imgs.reshape(B, H * W).astype(jnp.bfloat16)
    packed = lax.bitcast_convert_type(
        imgs_bf.reshape(B, (H * W) // 2, 2), jnp.int32)
    dvf_flat = dvfs.reshape(-1)
    out = _run(packed, dvf_flat)
    return out.reshape(B, H, W, C)


# speedup vs baseline: 2.8314x; 2.8314x over previous
import functools

import jax
import jax.numpy as jnp
from jax import lax
from jax.experimental import pallas as pl
from jax.experimental.pallas import tpu as pltpu
from jax.experimental.pallas import tpu_sc as plsc

_B, _H, _W = 8, 384, 384
_NPIX = _H * _W
_NWORDS = _NPIX // 2
_TILES_PER_IMG = 4
_ROWS_PER_TILE = _H // _TILES_PER_IMG
_CHUNK_ROWS = 8
_CHUNK_PIX = _CHUNK_ROWS * _W
_NCHUNKS = _ROWS_PER_TILE // _CHUNK_ROWS
_GROUPS_PER_ROW = _W // 16


def _sc_body(packed_ref, dvf_ref, out_ref, img_v, dvf_v, out_v):
    cid = lax.axis_index("c")
    sid = lax.axis_index("s")
    wid = sid * 2 + cid
    b = wid // _TILES_PER_IMG
    q = wid % _TILES_PER_IMG
    row0 = q * _ROWS_PER_TILE

    pltpu.sync_copy(packed_ref.at[b], img_v)

    lane = lax.iota(jnp.int32, 16)
    lanef = lane.astype(jnp.float32)

    for ch in range(_NCHUNKS):
        crow = row0 + ch * _CHUNK_ROWS
        dsrc = (b * _H + crow) * 2 * _W
        pltpu.sync_copy(dvf_ref.at[pl.ds(dsrc, _CHUNK_PIX * 2)], dvf_v)

        def _row(r, carry):
            rowf = (crow + r).astype(jnp.float32)

            def _grp(t, carry2):
                p0 = r * _W + t * 16
                doff = 2 * r * _W + t * 16
                dx = dvf_v[pl.ds(doff, 16)]
                dy = dvf_v[pl.ds(doff + _W, 16)]

                fx = (t * 16).astype(jnp.float32) + lanef + dx
                fy = rowf + dy
                x0 = fx.astype(jnp.int32)
                y0 = fy.astype(jnp.int32)
                x1 = x0 + 1
                y1 = y0 + 1
                x0 = jnp.clip(x0, 0, _W - 1)
                x1 = jnp.clip(x1, 0, _W - 1)
                y0 = jnp.clip(y0, 0, _H - 1)
                y1 = jnp.clip(y1, 0, _H - 1)

                ry0 = y0 * _W
                ry1 = y1 * _W
                pa = ry0 + x0
                pb = ry1 + x0
                pc = ry0 + x1
                pd = ry1 + x1

                def fetch(p):
                    word = plsc.load_gather(img_v, [p >> 1])
                    sh = ((p & 1) ^ 1) << 4
                    bits = (word << sh) & jnp.int32(-65536)
                    return plsc.bitcast(bits, jnp.float32)

                va = fetch(pa)
                vb = fetch(pb)
                vc = fetch(pc)
                vd = fetch(pd)

                x0f = x0.astype(jnp.float32)
                x1f = x1.astype(jnp.float32)
                y0f = y0.astype(jnp.float32)
                y1f = y1.astype(jnp.float32)
                wx1 = x1f - fx
                wx0 = fx - x0f
                wy1 = y1f - fy
                wy0 = fy - y0f
                res = ((wx1 * wy1) * va + (wx1 * wy0) * vb
                       + (wx0 * wy1) * vc + (wx0 * wy0) * vd)
                out_v[pl.ds(p0, 16)] = res
                return carry2

            return lax.fori_loop(0, _GROUPS_PER_ROW, _grp, carry)

        lax.fori_loop(0, _CHUNK_ROWS, _row, jnp.int32(0))

        dst = b * _NPIX + crow * _W
        pltpu.sync_copy(out_v, out_ref.at[pl.ds(dst, _CHUNK_PIX)])


@jax.jit
def _run(packed, dvf_flat):
    mesh = plsc.VectorSubcoreMesh(core_axis_name="c", subcore_axis_name="s")
    fn = pl.kernel(
        _sc_body,
        out_type=jax.ShapeDtypeStruct((_B * _NPIX,), jnp.float32),
        mesh=mesh,
        scratch_types=[
            pltpu.VMEM((_NWORDS,), jnp.int32),
            pltpu.VMEM((_CHUNK_PIX * 2,), jnp.float32),
            pltpu.VMEM((_CHUNK_PIX,), jnp.float32),
        ],
        compiler_params=pltpu.CompilerParams(needs_layout_passes=False),
    )
    return fn(packed, dvf_flat)


def kernel(imgs, dvfs):
    B, H, W, C = imgs.shape
    imgs_bf = imgs.reshape(B, H * W).astype(jnp.bfloat16)
    packed = lax.bitcast_convert_type(
        imgs_bf.reshape(B, (H * W) // 2, 2), jnp.int32)
    dvf_flat = jnp.transpose(dvfs, (0, 1, 3, 2)).reshape(-1)
    out = _run(packed, dvf_flat)
    return out.reshape(B, H, W, C)


# speedup vs baseline: 12.5364x; 4.4276x over previous
import functools

import jax
import jax.numpy as jnp
from jax import lax
from jax.experimental import pallas as pl
from jax.experimental.pallas import tpu as pltpu
from jax.experimental.pallas import tpu_sc as plsc

_B, _H, _W = 8, 384, 384
_NPIX = _H * _W
_NWORDS = _NPIX // 2
_TILES_PER_IMG = 4
_ROWS_PER_TILE = _H // _TILES_PER_IMG
_CHUNK_ROWS = 8
_CHUNK_PIX = _CHUNK_ROWS * _W
_NCHUNKS = _ROWS_PER_TILE // _CHUNK_ROWS
_GROUPS_PER_ROW = _W // 16


def _sc_body(packed_ref, dvf_ref, out_ref, img_v, dvf_v, out_v):
    cid = lax.axis_index("c")
    sid = lax.axis_index("s")
    wid = sid * 2 + cid
    b = wid // _TILES_PER_IMG
    q = wid % _TILES_PER_IMG
    row0 = q * _ROWS_PER_TILE

    pltpu.sync_copy(packed_ref.at[pl.ds(b * _NWORDS, _NWORDS)], img_v)

    lane = lax.iota(jnp.int32, 16)
    lanef = lane.astype(jnp.float32)

    for ch in range(_NCHUNKS):
        crow = row0 + ch * _CHUNK_ROWS
        dsrc = (b * _H + crow) * 2 * _W
        pltpu.sync_copy(dvf_ref.at[pl.ds(dsrc, _CHUNK_PIX * 2)], dvf_v)

        def _row(r, carry):
            rowf = (crow + r).astype(jnp.float32)

            def _grp(t, carry2):
                p0 = r * _W + t * 16
                doff = 2 * r * _W + t * 16
                dx = dvf_v[pl.ds(doff, 16)]
                dy = dvf_v[pl.ds(doff + _W, 16)]

                fx = (t * 16).astype(jnp.float32) + lanef + dx
                fy = rowf + dy
                x0 = fx.astype(jnp.int32)
                y0 = fy.astype(jnp.int32)
                x1 = x0 + 1
                y1 = y0 + 1
                x0 = jnp.clip(x0, 0, _W - 1)
                x1 = jnp.clip(x1, 0, _W - 1)
                y0 = jnp.clip(y0, 0, _H - 1)
                y1 = jnp.clip(y1, 0, _H - 1)

                ry0 = y0 * _W
                ry1 = y1 * _W
                pa = ry0 + x0
                pb = ry1 + x0
                pc = ry0 + x1
                pd = ry1 + x1

                def fetch(p):
                    is_lo = p < _NWORDS
                    widx = jnp.where(is_lo, p, p - _NWORDS)
                    sh = jnp.where(is_lo, 16, 0)
                    word = plsc.load_gather(img_v, [widx])
                    bits = (word << sh) & jnp.int32(-65536)
                    return plsc.bitcast(bits, jnp.float32)

                va = fetch(pa)
                vb = fetch(pb)
                vc = fetch(pc)
                vd = fetch(pd)

                x0f = x0.astype(jnp.float32)
                x1f = x1.astype(jnp.float32)
                y0f = y0.astype(jnp.float32)
                y1f = y1.astype(jnp.float32)
                wx1 = x1f - fx
                wx0 = fx - x0f
                wy1 = y1f - fy
                wy0 = fy - y0f
                res = ((wx1 * wy1) * va + (wx1 * wy0) * vb
                       + (wx0 * wy1) * vc + (wx0 * wy0) * vd)
                out_v[pl.ds(p0, 16)] = res
                return carry2

            return lax.fori_loop(0, _GROUPS_PER_ROW, _grp, carry)

        lax.fori_loop(0, _CHUNK_ROWS, _row, jnp.int32(0))

        dst = b * _NPIX + crow * _W
        pltpu.sync_copy(out_v, out_ref.at[pl.ds(dst, _CHUNK_PIX)])


@jax.jit
def _run(packed, dvf_flat):
    mesh = plsc.VectorSubcoreMesh(core_axis_name="c", subcore_axis_name="s")
    fn = pl.kernel(
        _sc_body,
        out_type=jax.ShapeDtypeStruct((_B * _NPIX,), jnp.float32),
        name="bilerp_sc",
        mesh=mesh,
        scratch_types=[
            pltpu.VMEM((_NWORDS,), jnp.int32),
            pltpu.VMEM((_CHUNK_PIX * 2,), jnp.float32),
            pltpu.VMEM((_CHUNK_PIX,), jnp.float32),
        ],
        compiler_params=pltpu.CompilerParams(needs_layout_passes=False),
    )
    return fn(packed, dvf_flat)


def kernel(imgs, dvfs):
    B, H, W, C = imgs.shape
    bits = lax.bitcast_convert_type(imgs.reshape(-1), jnp.uint32)
    rne = (bits + jnp.uint32(0x7FFF) + ((bits >> 16) & 1)) >> 16
    npix = H * W
    nw = npix // 2
    lo = jnp.concatenate([rne[s * npix: s * npix + nw] for s in range(B)])
    hi = jnp.concatenate([rne[s * npix + nw: (s + 1) * npix] for s in range(B)])
    packed = lax.bitcast_convert_type(lo | (hi << 16), jnp.int32)
    dvf_flat = jnp.transpose(dvfs, (0, 1, 3, 2)).reshape(-1)
    out = _run(packed, dvf_flat)
    return out.reshape(B, H, W, C)
